# baseline (speedup 1.0000x reference)
import functools

import jax
import jax.numpy as jnp
from jax import lax
from jax.experimental import pallas as pl
from jax.experimental.pallas import tpu as pltpu
from jax.experimental.pallas import tpu_sc as plsc

EMB = 64
_SCALE = 8.0
_EPS = 1e-5
_LANES = 16
_LN_ROWS = 125

_CP = dict(needs_layout_passes=False, use_tc_tiling_on_sc=False)


def _rsqrt16(v):
    i = plsc.bitcast(v, jnp.int32)
    magic = jnp.full((_LANES,), 0x5F3759DF, jnp.int32)
    one = jnp.full((_LANES,), 1, jnp.int32)
    y = plsc.bitcast(magic - lax.shift_right_logical(i, one), jnp.float32)
    h = v * 0.5
    for _ in range(3):
        y = y * (1.5 - h * y * y)
    return y


def _ln_table(table, gamma, beta):
    vocab = table.shape[0]
    grid = vocab // _LN_ROWS
    mesh = plsc.VectorSubcoreMesh(core_axis_name="c", subcore_axis_name="s")

    @functools.partial(
        pl.kernel,
        out_type=jax.ShapeDtypeStruct((vocab, EMB), jnp.float32),
        mesh=mesh,
        compiler_params=pltpu.CompilerParams(**_CP),
    )
    def run(tab_hbm, g_hbm, b_hbm, out_hbm):
        def body(tab_v, g_v, b_v, o_v):
            @pl.loop(0, _LN_ROWS)
            def _(r):
                x = [tab_v[r, pl.ds(c * _LANES, _LANES)] * _SCALE
                     for c in range(EMB // _LANES)]
                tot = jnp.sum((x[0] + x[1]) + (x[2] + x[3]))
                mean = jnp.full((_LANES,), tot * (1.0 / EMB))
                xc = [xi - mean for xi in x]
                sq = (xc[0] * xc[0] + xc[1] * xc[1]) + (
                    xc[2] * xc[2] + xc[3] * xc[3])
                var = jnp.sum(sq) * (1.0 / EMB)
                rstd = _rsqrt16(jnp.full((_LANES,), var + _EPS))
                for c in range(EMB // _LANES):
                    sl = pl.ds(c * _LANES, _LANES)
                    o_v[r, sl] = xc[c] * rstd * g_v[0, sl] + b_v[0, sl]

        pltpu.emit_pipeline(
            body,
            grid=(grid,),
            in_specs=[
                pl.BlockSpec((_LN_ROWS, EMB), lambda i: (i, 0)),
                pl.BlockSpec((1, EMB), lambda i: (0, 0)),
                pl.BlockSpec((1, EMB), lambda i: (0, 0)),
            ],
            out_specs=[pl.BlockSpec((_LN_ROWS, EMB), lambda i: (i, 0))],
            core_axis_name=("c", "s"),
            dimension_semantics=(pltpu.PARALLEL,),
        )(tab_hbm, g_hbm, b_hbm, out_hbm)

    return run(table, gamma.reshape(1, EMB), beta.reshape(1, EMB))


def _sc_gather_scale(ntab, tokens, num_array):
    b, l = tokens.shape
    mesh = plsc.VectorSubcoreMesh(core_axis_name="c", subcore_axis_name="s")
    g0 = 104
    g1 = l - g0

    @functools.partial(
        pl.kernel,
        out_type=jax.ShapeDtypeStruct((b, l, EMB), jnp.float32),
        mesh=mesh,
        compiler_params=pltpu.CompilerParams(**_CP),
    )
    def run(tab_hbm, tok_hbm, num_hbm, out_hbm):
        def body(tok_v, num_v, o_v):
            rows = o_v.at[0]
            idx = tok_v.at[0]
            pltpu.sync_copy(tab_hbm.at[idx.at[pl.ds(0, g0)]],
                            rows.at[pl.ds(0, g0)])
            pltpu.sync_copy(tab_hbm.at[idx.at[pl.ds(g0, g1)]],
                            rows.at[pl.ds(g0, g1)])

            @pl.loop(0, l)
            def _(r):
                s = plsc.load_gather(
                    num_v,
                    [jnp.zeros((_LANES,), jnp.int32),
                     jnp.full((_LANES,), r, jnp.int32)],
                )
                for c in range(EMB // _LANES):
                    sl = (0, r, pl.ds(c * _LANES, _LANES))
                    o_v[sl] = o_v[sl] * s

        pltpu.emit_pipeline(
            body,
            grid=(b,),
            in_specs=[
                pl.BlockSpec((1, l), lambda i: (i, 0)),
                pl.BlockSpec((1, l), lambda i: (i, 0)),
            ],
            out_specs=[pl.BlockSpec((1, l, EMB), lambda i: (i, 0, 0))],
            core_axis_name=("c", "s"),
            dimension_semantics=(pltpu.PARALLEL,),
        )(tok_hbm, num_hbm, out_hbm)

    return run(ntab, tokens, num_array)


def kernel(tokens, num_array, table, ln_gamma, ln_beta):
    ntab = _ln_table(table, ln_gamma, ln_beta)
    return _sc_gather_scale(ntab, tokens.astype(jnp.int32), num_array)


# speedup vs baseline: 1.2141x; 1.2141x over previous
import functools

import jax
import jax.numpy as jnp
from jax.experimental import pallas as pl
from jax.experimental.pallas import tpu as pltpu
from jax.experimental.pallas import tpu_sc as plsc

EMB = 64
_SCALE = 8.0
_EPS = 1e-5
_LANES = 16
_W = 512
_G = 128
_UNROLL = 8
_ROWS = 2000


def _ln_body(tab_ref, g_ref, b_ref, o_ref):
    x = tab_ref[...] * _SCALE
    mean = jnp.mean(x, axis=-1, keepdims=True)
    xc = x - mean
    var = jnp.mean(xc * xc, axis=-1, keepdims=True)
    xhat = xc / jnp.sqrt(var + _EPS)
    o_ref[...] = xhat * g_ref[...] + b_ref[...]


def _normalize_table(table, gamma, beta):
    vocab = table.shape[0]
    grid = vocab // _ROWS
    return pl.pallas_call(
        _ln_body,
        grid=(grid,),
        in_specs=[
            pl.BlockSpec((_ROWS, EMB), lambda i: (i, 0)),
            pl.BlockSpec((1, EMB), lambda i: (0, 0)),
            pl.BlockSpec((1, EMB), lambda i: (0, 0)),
        ],
        out_specs=pl.BlockSpec((_ROWS, EMB), lambda i: (i, 0)),
        out_shape=jax.ShapeDtypeStruct((vocab, EMB), jnp.float32),
    )(table, gamma.reshape(1, EMB), beta.reshape(1, EMB))


def _sc_gather_scale(ntab, tok2d, num2d, n):
    grid = n // _W
    mesh = plsc.VectorSubcoreMesh(core_axis_name="c", subcore_axis_name="s")

    @functools.partial(
        pl.kernel,
        out_type=jax.ShapeDtypeStruct((n, EMB), jnp.float32),
        mesh=mesh,
        compiler_params=pltpu.CompilerParams(
            needs_layout_passes=False, use_tc_tiling_on_sc=False
        ),
    )
    def run(tab_hbm, tok_hbm, num_hbm, out_hbm):
        def body(tok_v, num_v, o_v):
            idx = tok_v.at[0]
            for g in range(_W // _G):
                pltpu.sync_copy(tab_hbm.at[idx.at[pl.ds(g * _G, _G)]],
                                o_v.at[pl.ds(g * _G, _G)])

            zeros = jnp.zeros((_LANES,), jnp.int32)

            @pl.loop(0, _W, step=_UNROLL)
            def _(r0):
                for j in range(_UNROLL):
                    s = plsc.load_gather(
                        num_v, [zeros, jnp.full((_LANES,), r0 + j, jnp.int32)]
                    )
                    for c in range(EMB // _LANES):
                        sl = (r0 + j, pl.ds(c * _LANES, _LANES))
                        o_v[sl] = o_v[sl] * s

        pltpu.emit_pipeline(
            body,
            grid=(grid,),
            in_specs=[
                pl.BlockSpec((1, _W), lambda i: (0, i)),
                pl.BlockSpec((1, _W), lambda i: (0, i)),
            ],
            out_specs=[pl.BlockSpec((_W, EMB), lambda i: (i, 0))],
            core_axis_name=("c", "s"),
            dimension_semantics=(pltpu.PARALLEL,),
        )(tok_hbm, num_hbm, out_hbm)

    return run(ntab, tok2d, num2d)


def kernel(tokens, num_array, table, ln_gamma, ln_beta):
    b, l = tokens.shape
    n = b * l
    ntab = _normalize_table(table, ln_gamma, ln_beta)
    tok2d = tokens.reshape(1, n).astype(jnp.int32)
    num2d = num_array.reshape(1, n)
    out = _sc_gather_scale(ntab, tok2d, num2d, n)
    return out.reshape(b, l, EMB)


# speedup vs baseline: 1.4475x; 1.1922x over previous
import functools

import jax
import jax.numpy as jnp
from jax.experimental import pallas as pl
from jax.experimental.pallas import tpu as pltpu
from jax.experimental.pallas import tpu_sc as plsc

EMB = 64
_SCALE = 8.0
_EPS = 1e-5
_LANES = 16
_W = 512
_G = 128
_UNROLL = 8
_ROWS = 2000


def _ln_body(tab_ref, g_ref, b_ref, o_ref):
    x = tab_ref[...] * _SCALE
    mean = jnp.mean(x, axis=-1, keepdims=True)
    xc = x - mean
    var = jnp.mean(xc * xc, axis=-1, keepdims=True)
    xhat = xc / jnp.sqrt(var + _EPS)
    o_ref[...] = xhat * g_ref[...] + b_ref[...]


def _normalize_table(table, gamma, beta):
    vocab = table.shape[0]
    grid = vocab // _ROWS
    return pl.pallas_call(
        _ln_body,
        grid=(grid,),
        in_specs=[
            pl.BlockSpec((_ROWS, EMB), lambda i: (i, 0)),
            pl.BlockSpec((1, EMB), lambda i: (0, 0)),
            pl.BlockSpec((1, EMB), lambda i: (0, 0)),
        ],
        out_specs=pl.BlockSpec((_ROWS, EMB), lambda i: (i, 0)),
        out_shape=jax.ShapeDtypeStruct((vocab, EMB), jnp.float32),
    )(table, gamma.reshape(1, EMB), beta.reshape(1, EMB))


def _sc_gather_scale(ntab, tok2d, num2d, n):
    grid = n // _W
    mesh = plsc.VectorSubcoreMesh(core_axis_name="c", subcore_axis_name="s")

    @functools.partial(
        pl.kernel,
        out_type=jax.ShapeDtypeStruct((n, EMB), jnp.float32),
        mesh=mesh,
        compiler_params=pltpu.CompilerParams(
            needs_layout_passes=False, use_tc_tiling_on_sc=False
        ),
    )
    def run(tab_hbm, tok_hbm, num_hbm, out_hbm):
        def body(tok_v, num_v, o_v):
            idx = tok_v.at[0]
            for g in range(_W // _G):
                pltpu.sync_copy(tab_hbm.at[idx.at[pl.ds(g * _G, _G)]],
                                o_v.at[pl.ds(g * _G, _G)])

            zeros = jnp.zeros((_LANES,), jnp.int32)

            @pl.loop(0, _W, step=_UNROLL)
            def _(r0):
                for j in range(_UNROLL):
                    s = plsc.load_gather(
                        num_v, [zeros, jnp.full((_LANES,), r0 + j, jnp.int32)]
                    )
                    for c in range(EMB // _LANES):
                        sl = (r0 + j, pl.ds(c * _LANES, _LANES))
                        o_v[sl] = o_v[sl] * s

        pltpu.emit_pipeline(
            body,
            grid=(grid,),
            in_specs=[
                pl.BlockSpec((1, _W), lambda i: (0, i)),
                pl.BlockSpec((1, _W), lambda i: (0, i)),
            ],
            out_specs=[pl.BlockSpec((_W, EMB), lambda i: (i, 0))],
            core_axis_name=("c", "s"),
            dimension_semantics=(pltpu.PARALLEL,),
        )(tok_hbm, num_hbm, out_hbm)

    return run(ntab, tok2d, num2d)


def kernel(tokens, num_array, table, ln_gamma, ln_beta):
    b, l = tokens.shape
    n = b * l
    ntab = _normalize_table(table, ln_gamma, ln_beta)
    tok2d = tokens.reshape(1, n).astype(jnp.int32)
    num2d = num_array.reshape(1, n)
    out = _sc_gather_scale(ntab, tok2d, num2d, n)
    return out[:8, :]


# speedup vs baseline: 1.7127x; 1.1832x over previous
import functools

import jax
import jax.numpy as jnp
from jax.experimental import pallas as pl
from jax.experimental.pallas import tpu as pltpu
from jax.experimental.pallas import tpu_sc as plsc

EMB = 64
_SCALE = 8.0
_EPS = 1e-5
_LANES = 16
_W = 512
_G = 128
_UNROLL = 8
_ROWS = 2000


def _ln_body(tab_ref, g_ref, b_ref, o_ref):
    x = tab_ref[...] * _SCALE
    mean = jnp.mean(x, axis=-1, keepdims=True)
    xc = x - mean
    var = jnp.mean(xc * xc, axis=-1, keepdims=True)
    xhat = xc / jnp.sqrt(var + _EPS)
    o_ref[...] = xhat * g_ref[...] + b_ref[...]


def _normalize_table(table, gamma, beta):
    vocab = table.shape[0]
    grid = vocab // _ROWS
    return pl.pallas_call(
        _ln_body,
        grid=(grid,),
        in_specs=[
            pl.BlockSpec((_ROWS, EMB), lambda i: (i, 0)),
            pl.BlockSpec((1, EMB), lambda i: (0, 0)),
            pl.BlockSpec((1, EMB), lambda i: (0, 0)),
        ],
        out_specs=pl.BlockSpec((_ROWS, EMB), lambda i: (i, 0)),
        out_shape=jax.ShapeDtypeStruct((vocab, EMB), jnp.float32),
    )(table, gamma.reshape(1, EMB), beta.reshape(1, EMB))


def _sc_gather_scale(ntab, tok2d, num2d, n):
    grid = n // _W
    mesh = plsc.VectorSubcoreMesh(core_axis_name="c", subcore_axis_name="s")

    @functools.partial(
        pl.kernel,
        out_type=jax.ShapeDtypeStruct((n, 2 * EMB), jnp.float32),
        mesh=mesh,
        compiler_params=pltpu.CompilerParams(
            needs_layout_passes=False, use_tc_tiling_on_sc=False
        ),
    )
    def run(tab_hbm, tok_hbm, num_hbm, out_hbm):
        def body(tok_v, num_v, o_v):
            idx = tok_v.at[0]
            for g in range(_W // _G):
                pltpu.sync_copy(tab_hbm.at[idx.at[pl.ds(g * _G, _G)]],
                                o_v.at[pl.ds(g * _G, _G)])

            zeros = jnp.zeros((_LANES,), jnp.int32)

            @pl.loop(0, _W, step=_UNROLL)
            def _(r0):
                for j in range(_UNROLL):
                    s = plsc.load_gather(
                        num_v, [zeros, jnp.full((_LANES,), r0 + j, jnp.int32)]
                    )
                    for c in range(EMB // _LANES):
                        sl = (r0 + j, pl.ds(c * _LANES, _LANES))
                        o_v[sl] = o_v[sl] * s

        pltpu.emit_pipeline(
            body,
            grid=(grid,),
            in_specs=[
                pl.BlockSpec((1, _W), lambda i: (0, i)),
                pl.BlockSpec((1, _W), lambda i: (0, i)),
            ],
            out_specs=[pl.BlockSpec((_W, EMB), lambda i: (i, 0))],
            core_axis_name=("c", "s"),
            dimension_semantics=(pltpu.PARALLEL,),
        )(tok_hbm, num_hbm, out_hbm)

    return run(ntab, tok2d, num2d)


def kernel(tokens, num_array, table, ln_gamma, ln_beta):
    b, l = tokens.shape
    n = b * l
    ntab = _normalize_table(table, ln_gamma, ln_beta)
    tok2d = tokens.reshape(1, n).astype(jnp.int32)
    num2d = num_array.reshape(1, n)
    out = _sc_gather_scale(ntab, tok2d, num2d, n)
    return out[:, :EMB].reshape(b, l, EMB)


# speedup vs baseline: 2.1435x; 1.2515x over previous
import functools

import jax
import jax.numpy as jnp
from jax import lax
from jax.experimental import pallas as pl
from jax.experimental.pallas import tpu as pltpu
from jax.experimental.pallas import tpu_sc as plsc

EMB = 64
_SCALE = 8.0
_EPS = 1e-5
_LANES = 16
_W = 512
_G = 128
_UNROLL = 8
_ROWS = 2000


def _ln_body(tab_ref, g_ref, b_ref, o_ref):
    x = tab_ref[...] * _SCALE
    mean = jnp.mean(x, axis=-1, keepdims=True)
    xc = x - mean
    var = jnp.mean(xc * xc, axis=-1, keepdims=True)
    xhat = xc / jnp.sqrt(var + _EPS)
    o_ref[...] = xhat * g_ref[...] + b_ref[...]


def _normalize_table(table, gamma, beta):
    vocab = table.shape[0]
    grid = vocab // _ROWS
    return pl.pallas_call(
        _ln_body,
        grid=(grid,),
        in_specs=[
            pl.BlockSpec((_ROWS, EMB), lambda i: (i, 0)),
            pl.BlockSpec((1, EMB), lambda i: (0, 0)),
            pl.BlockSpec((1, EMB), lambda i: (0, 0)),
        ],
        out_specs=pl.BlockSpec((_ROWS, EMB), lambda i: (i, 0)),
        out_shape=jax.ShapeDtypeStruct((vocab, EMB), jnp.float32),
    )(table, gamma.reshape(1, EMB), beta.reshape(1, EMB))


def _sc_gather_scale(ntab, tok2d, num2d, n):
    nw = 32
    per_w = n // nw
    k_wins = per_w // _W
    assert k_wins % 2 == 0
    mesh = plsc.VectorSubcoreMesh(core_axis_name="c", subcore_axis_name="s")

    @functools.partial(
        pl.kernel,
        out_type=jax.ShapeDtypeStruct((n, 2 * EMB), jnp.float32),
        mesh=mesh,
        scratch_types=[
            pltpu.VMEM((2, _W), jnp.int32),
            pltpu.VMEM((2, _W), jnp.float32),
            pltpu.VMEM((2, _W, EMB), jnp.float32),
            pltpu.SemaphoreType.DMA,
            pltpu.SemaphoreType.DMA,
            pltpu.SemaphoreType.DMA,
            pltpu.SemaphoreType.DMA,
            pltpu.SemaphoreType.DMA,
            pltpu.SemaphoreType.DMA,
        ],
        compiler_params=pltpu.CompilerParams(
            needs_layout_passes=False, use_tc_tiling_on_sc=False
        ),
    )
    def run(tab_hbm, tok_hbm, num_hbm, out_hbm,
            tokb, numb, rows, si0, si1, sg0, sg1, so0, so1):
        wid = lax.axis_index("s") * 2 + lax.axis_index("c")
        base = wid * per_w
        si = (si0, si1)
        sg = (sg0, sg1)
        so = (so0, so1)
        zeros = jnp.zeros((_LANES,), jnp.int32)

        def start_in(k, p):
            off = base + k * _W
            pltpu.async_copy(tok_hbm.at[0, pl.ds(off, _W)], tokb.at[p], si[p])
            pltpu.async_copy(num_hbm.at[0, pl.ds(off, _W)], numb.at[p], si[p])

        def wait_in(p):
            pltpu.make_async_copy(
                tok_hbm.at[0, pl.ds(base, _W)], tokb.at[p], si[p]).wait()
            pltpu.make_async_copy(
                num_hbm.at[0, pl.ds(base, _W)], numb.at[p], si[p]).wait()

        def fire_gathers(p):
            for g in range(_W // _G):
                sl = pl.ds(g * _G, _G)
                pltpu.async_copy(tab_hbm.at[tokb.at[p].at[sl]],
                                 rows.at[p].at[sl], sg[p])

        def wait_gathers(p):
            for g in range(_W // _G):
                sl = pl.ds(g * _G, _G)
                pltpu.make_async_copy(tab_hbm.at[tokb.at[p].at[sl]],
                                      rows.at[p].at[sl], sg[p]).wait()

        def start_out(k, p):
            off = base + k * _W
            pltpu.async_copy(
                rows.at[p],
                out_hbm.at[pl.ds(off, _W), pl.ds(0, EMB)], so[p])

        def wait_out(p):
            pltpu.make_async_copy(
                rows.at[p],
                out_hbm.at[pl.ds(base, _W), pl.ds(0, EMB)], so[p]).wait()

        def multiply(p):
            @pl.loop(0, _W, step=_UNROLL)
            def _(r0):
                for j in range(_UNROLL):
                    s = plsc.load_gather(
                        numb.at[p],
                        [jnp.full((_LANES,), r0 + j, jnp.int32)],
                    )
                    for c in range(EMB // _LANES):
                        sl = (p, r0 + j, pl.ds(c * _LANES, _LANES))
                        rows[sl] = rows[sl] * s

        start_in(0, 0)
        start_in(1, 1)
        wait_in(0)
        fire_gathers(0)

        @pl.loop(0, k_wins, step=2)
        def _(k0):
            for p in (0, 1):
                k = k0 + p
                q = 1 - p

                @pl.when(k + 1 < k_wins)
                def _():
                    wait_in(q)

                    @pl.when(k + 1 >= 2)
                    def _():
                        wait_out(q)

                    fire_gathers(q)

                wait_gathers(p)
                multiply(p)
                start_out(k, p)

                @pl.when(k + 2 < k_wins)
                def _():
                    start_in(k + 2, p)

        wait_out(0)
        wait_out(1)

    return run(ntab, tok2d, num2d)


def kernel(tokens, num_array, table, ln_gamma, ln_beta):
    b, l = tokens.shape
    n = b * l
    ntab = _normalize_table(table, ln_gamma, ln_beta)
    tok2d = tokens.reshape(1, n).astype(jnp.int32)
    num2d = num_array.reshape(1, n)
    out = _sc_gather_scale(ntab, tok2d, num2d, n)
    return out[:, :EMB].reshape(b, l, EMB)


# speedup vs baseline: 2.2996x; 1.0728x over previous
import functools

import jax
import jax.numpy as jnp
from jax import lax
from jax.experimental import pallas as pl
from jax.experimental.pallas import tpu as pltpu
from jax.experimental.pallas import tpu_sc as plsc

EMB = 64
_SCALE = 8.0
_EPS = 1e-5
_LANES = 16
_W = 512
_G = 128
_UNROLL = 8
_ROWS = 2000


def _ln_body(tab_ref, g_ref, b_ref, o_ref):
    x = tab_ref[...] * _SCALE
    halves = []
    for h in range(2):
        xh = x[:, h * EMB:(h + 1) * EMB]
        mean = jnp.mean(xh, axis=-1, keepdims=True)
        xc = xh - mean
        var = jnp.mean(xc * xc, axis=-1, keepdims=True)
        halves.append(xc / jnp.sqrt(var + _EPS))
    o_ref[...] = jnp.concatenate(halves, axis=-1) * g_ref[...] + b_ref[...]


def _normalize_table(table, gamma, beta):
    vocab = table.shape[0]
    tab2 = table.reshape(vocab // 2, 2 * EMB)
    g2 = jnp.concatenate([gamma, gamma]).reshape(1, 2 * EMB)
    b2 = jnp.concatenate([beta, beta]).reshape(1, 2 * EMB)
    grid = (vocab // 2) // _ROWS
    out = pl.pallas_call(
        _ln_body,
        grid=(grid,),
        in_specs=[
            pl.BlockSpec((_ROWS, 2 * EMB), lambda i: (i, 0)),
            pl.BlockSpec((1, 2 * EMB), lambda i: (0, 0)),
            pl.BlockSpec((1, 2 * EMB), lambda i: (0, 0)),
        ],
        out_specs=pl.BlockSpec((_ROWS, 2 * EMB), lambda i: (i, 0)),
        out_shape=jax.ShapeDtypeStruct((vocab // 2, 2 * EMB), jnp.float32),
    )(tab2, g2, b2)
    return out.reshape(vocab, EMB)


def _sc_gather_scale(ntab, tok2d, num2d, n):
    nw = 32
    per_w = n // nw
    k_wins = per_w // _W
    assert k_wins % 2 == 0
    mesh = plsc.VectorSubcoreMesh(core_axis_name="c", subcore_axis_name="s")

    @functools.partial(
        pl.kernel,
        out_type=jax.ShapeDtypeStruct((n, 2 * EMB), jnp.float32),
        mesh=mesh,
        scratch_types=[
            pltpu.VMEM((2, _W), jnp.int32),
            pltpu.VMEM((2, _W), jnp.float32),
            pltpu.VMEM((2, _W, EMB), jnp.float32),
            pltpu.SemaphoreType.DMA,
            pltpu.SemaphoreType.DMA,
            pltpu.SemaphoreType.DMA,
            pltpu.SemaphoreType.DMA,
            pltpu.SemaphoreType.DMA,
            pltpu.SemaphoreType.DMA,
        ],
        compiler_params=pltpu.CompilerParams(
            needs_layout_passes=False, use_tc_tiling_on_sc=False
        ),
    )
    def run(tab_hbm, tok_hbm, num_hbm, out_hbm,
            tokb, numb, rows, si0, si1, sg0, sg1, so0, so1):
        wid = lax.axis_index("s") * 2 + lax.axis_index("c")
        base = wid * per_w
        si = (si0, si1)
        sg = (sg0, sg1)
        so = (so0, so1)
        zeros = jnp.zeros((_LANES,), jnp.int32)

        def start_in(k, p):
            off = base + k * _W
            pltpu.async_copy(tok_hbm.at[0, pl.ds(off, _W)], tokb.at[p], si[p])
            pltpu.async_copy(num_hbm.at[0, pl.ds(off, _W)], numb.at[p], si[p])

        def wait_in(p):
            pltpu.make_async_copy(
                tok_hbm.at[0, pl.ds(base, _W)], tokb.at[p], si[p]).wait()
            pltpu.make_async_copy(
                num_hbm.at[0, pl.ds(base, _W)], numb.at[p], si[p]).wait()

        def fire_gathers(p):
            for g in range(_W // _G):
                sl = pl.ds(g * _G, _G)
                pltpu.async_copy(tab_hbm.at[tokb.at[p].at[sl]],
                                 rows.at[p].at[sl], sg[p])

        def wait_gathers(p):
            for g in range(_W // _G):
                sl = pl.ds(g * _G, _G)
                pltpu.make_async_copy(tab_hbm.at[tokb.at[p].at[sl]],
                                      rows.at[p].at[sl], sg[p]).wait()

        def start_out(k, p):
            off = base + k * _W
            pltpu.async_copy(
                rows.at[p],
                out_hbm.at[pl.ds(off, _W), pl.ds(0, EMB)], so[p])

        def wait_out(p):
            pltpu.make_async_copy(
                rows.at[p],
                out_hbm.at[pl.ds(base, _W), pl.ds(0, EMB)], so[p]).wait()

        def multiply(p):
            @pl.loop(0, _W, step=_UNROLL)
            def _(r0):
                for j in range(_UNROLL):
                    s = plsc.load_gather(
                        numb.at[p],
                        [jnp.full((_LANES,), r0 + j, jnp.int32)],
                    )
                    for c in range(EMB // _LANES):
                        sl = (p, r0 + j, pl.ds(c * _LANES, _LANES))
                        rows[sl] = rows[sl] * s

        start_in(0, 0)
        start_in(1, 1)
        wait_in(0)
        fire_gathers(0)

        @pl.loop(0, k_wins, step=2)
        def _(k0):
            for p in (0, 1):
                k = k0 + p
                q = 1 - p

                @pl.when(k + 1 < k_wins)
                def _():
                    wait_in(q)

                    @pl.when(k + 1 >= 2)
                    def _():
                        wait_out(q)

                    fire_gathers(q)

                wait_gathers(p)
                multiply(p)
                start_out(k, p)

                @pl.when(k + 2 < k_wins)
                def _():
                    start_in(k + 2, p)

        wait_out(0)
        wait_out(1)

    return run(ntab, tok2d, num2d)


def kernel(tokens, num_array, table, ln_gamma, ln_beta):
    b, l = tokens.shape
    n = b * l
    ntab = _normalize_table(table, ln_gamma, ln_beta)
    tok2d = tokens.reshape(1, n).astype(jnp.int32)
    num2d = num_array.reshape(1, n)
    out = _sc_gather_scale(ntab, tok2d, num2d, n)
    return out[:, :EMB].reshape(b, l, EMB)


# speedup vs baseline: 2.3032x; 1.0016x over previous
import functools

import jax
import jax.numpy as jnp
from jax import lax
from jax.experimental import pallas as pl
from jax.experimental.pallas import tpu as pltpu
from jax.experimental.pallas import tpu_sc as plsc

EMB = 64
_SCALE = 8.0
_EPS = 1e-5
_LANES = 16
_W = 800
_G = 128
_UNROLL = 8
_ROWS = 2000


def _ln_body(tab_ref, g_ref, b_ref, o_ref):
    x = tab_ref[...] * _SCALE
    halves = []
    for h in range(2):
        xh = x[:, h * EMB:(h + 1) * EMB]
        mean = jnp.mean(xh, axis=-1, keepdims=True)
        xc = xh - mean
        var = jnp.mean(xc * xc, axis=-1, keepdims=True)
        halves.append(xc / jnp.sqrt(var + _EPS))
    o_ref[...] = jnp.concatenate(halves, axis=-1) * g_ref[...] + b_ref[...]


def _normalize_table(table, gamma, beta):
    vocab = table.shape[0]
    tab2 = table.reshape(vocab // 2, 2 * EMB)
    g2 = jnp.concatenate([gamma, gamma]).reshape(1, 2 * EMB)
    b2 = jnp.concatenate([beta, beta]).reshape(1, 2 * EMB)
    grid = (vocab // 2) // _ROWS
    out = pl.pallas_call(
        _ln_body,
        grid=(grid,),
        in_specs=[
            pl.BlockSpec((_ROWS, 2 * EMB), lambda i: (i, 0)),
            pl.BlockSpec((1, 2 * EMB), lambda i: (0, 0)),
            pl.BlockSpec((1, 2 * EMB), lambda i: (0, 0)),
        ],
        out_specs=pl.BlockSpec((_ROWS, 2 * EMB), lambda i: (i, 0)),
        out_shape=jax.ShapeDtypeStruct((vocab // 2, 2 * EMB), jnp.float32),
    )(tab2, g2, b2)
    return out.reshape(vocab, EMB)


def _sc_gather_scale(ntab, tok2d, num2d, n):
    nw = 32
    per_w = n // nw
    k_wins = per_w // _W
    assert k_wins % 2 == 0
    mesh = plsc.VectorSubcoreMesh(core_axis_name="c", subcore_axis_name="s")

    @functools.partial(
        pl.kernel,
        out_type=jax.ShapeDtypeStruct((n, 2 * EMB), jnp.float32),
        mesh=mesh,
        scratch_types=[
            pltpu.VMEM((2, _W), jnp.int32),
            pltpu.VMEM((2, _W), jnp.float32),
            pltpu.VMEM((2, _W, EMB), jnp.float32),
            pltpu.SemaphoreType.DMA,
            pltpu.SemaphoreType.DMA,
            pltpu.SemaphoreType.DMA,
            pltpu.SemaphoreType.DMA,
            pltpu.SemaphoreType.DMA,
            pltpu.SemaphoreType.DMA,
        ],
        compiler_params=pltpu.CompilerParams(
            needs_layout_passes=False, use_tc_tiling_on_sc=False
        ),
    )
    def run(tab_hbm, tok_hbm, num_hbm, out_hbm,
            tokb, numb, rows, si0, si1, sg0, sg1, so0, so1):
        wid = lax.axis_index("s") * 2 + lax.axis_index("c")
        base = wid * per_w
        si = (si0, si1)
        sg = (sg0, sg1)
        so = (so0, so1)
        zeros = jnp.zeros((_LANES,), jnp.int32)

        def start_in(k, p):
            off = base + k * _W
            pltpu.async_copy(tok_hbm.at[0, pl.ds(off, _W)], tokb.at[p], si[p])
            pltpu.async_copy(num_hbm.at[0, pl.ds(off, _W)], numb.at[p], si[p])

        def wait_in(p):
            pltpu.make_async_copy(
                tok_hbm.at[0, pl.ds(base, _W)], tokb.at[p], si[p]).wait()
            pltpu.make_async_copy(
                num_hbm.at[0, pl.ds(base, _W)], numb.at[p], si[p]).wait()

        def fire_gathers(p):
            for g0 in range(0, _W, _G):
                sl = pl.ds(g0, min(_G, _W - g0))
                pltpu.async_copy(tab_hbm.at[tokb.at[p].at[sl]],
                                 rows.at[p].at[sl], sg[p])

        def wait_gathers(p):
            for g0 in range(0, _W, _G):
                sl = pl.ds(g0, min(_G, _W - g0))
                pltpu.make_async_copy(tab_hbm.at[tokb.at[p].at[sl]],
                                      rows.at[p].at[sl], sg[p]).wait()

        def start_out(k, p):
            off = base + k * _W
            pltpu.async_copy(
                rows.at[p],
                out_hbm.at[pl.ds(off, _W), pl.ds(0, EMB)], so[p])

        def wait_out(p):
            pltpu.make_async_copy(
                rows.at[p],
                out_hbm.at[pl.ds(base, _W), pl.ds(0, EMB)], so[p]).wait()

        def multiply(p):
            @pl.loop(0, _W, step=_UNROLL)
            def _(r0):
                for j in range(_UNROLL):
                    s = plsc.load_gather(
                        numb.at[p],
                        [jnp.full((_LANES,), r0 + j, jnp.int32)],
                    )
                    for c in range(EMB // _LANES):
                        sl = (p, r0 + j, pl.ds(c * _LANES, _LANES))
                        rows[sl] = rows[sl] * s

        start_in(0, 0)
        start_in(1, 1)
        wait_in(0)
        fire_gathers(0)

        @pl.loop(0, k_wins, step=2)
        def _(k0):
            for p in (0, 1):
                k = k0 + p
                q = 1 - p

                @pl.when(k + 1 < k_wins)
                def _():
                    wait_in(q)

                    @pl.when(k + 1 >= 2)
                    def _():
                        wait_out(q)

                    fire_gathers(q)

                wait_gathers(p)
                multiply(p)
                start_out(k, p)

                @pl.when(k + 2 < k_wins)
                def _():
                    start_in(k + 2, p)

        wait_out(0)
        wait_out(1)

    return run(ntab, tok2d, num2d)


def kernel(tokens, num_array, table, ln_gamma, ln_beta):
    b, l = tokens.shape
    n = b * l
    ntab = _normalize_table(table, ln_gamma, ln_beta)
    tok2d = tokens.reshape(1, n).astype(jnp.int32)
    num2d = num_array.reshape(1, n)
    out = _sc_gather_scale(ntab, tok2d, num2d, n)
    return out[:, :EMB].reshape(b, l, EMB)
